# baseline (speedup 1.0000x reference)
import functools

import jax
import jax.numpy as jnp
from jax import lax
from jax.experimental import pallas as pl
from jax.experimental.pallas import tpu as pltpu
from jax.experimental.pallas import tpu_sc as plsc

NC = 2
NS = 16
NW = NC * NS
K = 128


def _make_sc_scatter(N, C, n_pad, nch):
    rpt = n_pad // NS
    zch = rpt // 8

    mesh = plsc.VectorSubcoreMesh(core_axis_name="c", subcore_axis_name="s")

    @functools.partial(
        pl.kernel,
        mesh=mesh,
        out_type=jax.ShapeDtypeStruct((NC, n_pad, C), jnp.float32),
        scratch_types=[
            pltpu.VMEM((nch, K), jnp.int32),
            pltpu.VMEM((nch, K), jnp.int32),
            pltpu.VMEM((K, C), jnp.float32),
            pltpu.VMEM((zch, C), jnp.float32),
            pltpu.VMEM_SHARED((n_pad, C), jnp.float32),
        ],
    )
    def sc_scatter(h_hbm, src_hbm, dst_hbm, out_hbm,
                   src_v, dst_v, buf, zbuf, m_sh):
        c = lax.axis_index("c")
        s = lax.axis_index("s")
        wid = c * NS + s

        pltpu.sync_copy(src_hbm.at[wid], src_v)
        pltpu.sync_copy(dst_hbm.at[wid], dst_v)

        @pl.loop(0, zch)
        def _(r):
            @pl.loop(0, C, step=16)
            def _(cc):
                zbuf.at[r, pl.ds(cc, 16)][...] = jnp.zeros((16,), jnp.float32)

        @pl.loop(0, 8)
        def _(k):
            pltpu.sync_copy(zbuf, m_sh.at[pl.ds(s * rpt + k * zch, zch)])

        plsc.subcore_barrier()

        @pl.loop(0, nch)
        def _(j):
            pltpu.sync_copy(h_hbm.at[src_v.at[j]], buf)
            pltpu.sync_copy(buf, m_sh.at[dst_v.at[j]], add=True)

        plsc.subcore_barrier()
        pltpu.sync_copy(m_sh.at[pl.ds(s * rpt, rpt)],
                        out_hbm.at[c, pl.ds(s * rpt, rpt)])

    return sc_scatter


def _gru_body(p_ref, h_ref, w_ref, wih_ref, whh_ref, bih_ref, bhh_ref, o_ref):
    C = h_ref.shape[1]
    sacc = p_ref[0] + p_ref[1]
    m = jnp.dot(sacc, w_ref[...], preferred_element_type=jnp.float32)
    gi = lax.dot_general(m, wih_ref[...], (((1,), (1,)), ((), ())),
                         preferred_element_type=jnp.float32) + bih_ref[...]
    gh = lax.dot_general(h_ref[...], whh_ref[...], (((1,), (1,)), ((), ())),
                         preferred_element_type=jnp.float32) + bhh_ref[...]
    i_r, i_z, i_n = gi[:, :C], gi[:, C:2 * C], gi[:, 2 * C:]
    h_r, h_z, h_n = gh[:, :C], gh[:, C:2 * C], gh[:, 2 * C:]
    r = jax.nn.sigmoid(i_r + h_r)
    z = jax.nn.sigmoid(i_z + h_z)
    n = jnp.tanh(i_n + r * h_n)
    o_ref[...] = (1.0 - z) * n + z * h_ref[...]


def _final_body(h_ref, wl_ref, bl_ref, wa_ref, o_ref):
    hr = jnp.maximum(h_ref[...], 0.0)
    h2 = lax.dot_general(hr, wl_ref[...], (((1,), (1,)), ((), ())),
                         preferred_element_type=jnp.float32) + bl_ref[...]
    logit = lax.dot_general(h2, wa_ref[...], (((1,), (1,)), ((), ())),
                            preferred_element_type=jnp.float32)
    mx = jnp.max(logit)
    e = jnp.exp(logit - mx)
    zsum = jnp.sum(e)
    o_ref[...] = jnp.sum(e * h2, axis=0, keepdims=True) / zsum


def kernel(x, edge_index, weight, W_ih, W_hh, b_ih, b_hh, W_lin, b_lin, W_att, b_att):
    N, C = x.shape
    E = edge_index.shape[1]
    L = weight.shape[0]

    ew = -(-E // (NW * K)) * K
    e_pad = ew * NW
    nch = ew // K
    n_pad = -(-(N + 1) // (NS * 8)) * (NS * 8)

    src = edge_index[0]
    dst = edge_index[1]
    perm = jnp.argsort(src)
    src = src[perm]
    dst = dst[perm]
    if e_pad > E:
        pad = e_pad - E
        src = jnp.concatenate([src, jnp.zeros((pad,), jnp.int32)])
        dst = jnp.concatenate([dst, jnp.full((pad,), N, jnp.int32)])
    srcr = src.reshape(NW, nch, K)
    dstr = dst.reshape(NW, nch, K)

    sc_scatter = _make_sc_scatter(N, C, n_pad, nch)

    BN = 1000
    gru = pl.pallas_call(
        _gru_body,
        grid=(N // BN,),
        in_specs=[
            pl.BlockSpec((NC, BN, C), lambda i: (0, i, 0)),
            pl.BlockSpec((BN, C), lambda i: (i, 0)),
            pl.BlockSpec((C, C), lambda i: (0, 0)),
            pl.BlockSpec((3 * C, C), lambda i: (0, 0)),
            pl.BlockSpec((3 * C, C), lambda i: (0, 0)),
            pl.BlockSpec((1, 3 * C), lambda i: (0, 0)),
            pl.BlockSpec((1, 3 * C), lambda i: (0, 0)),
        ],
        out_specs=pl.BlockSpec((BN, C), lambda i: (i, 0)),
        out_shape=jax.ShapeDtypeStruct((N, C), jnp.float32),
    )

    final = pl.pallas_call(
        _final_body,
        in_specs=[
            pl.BlockSpec((N, C), lambda: (0, 0)),
            pl.BlockSpec((C, C), lambda: (0, 0)),
            pl.BlockSpec((1, C), lambda: (0, 0)),
            pl.BlockSpec((1, C), lambda: (0, 0)),
        ],
        out_specs=pl.BlockSpec((1, C), lambda: (0, 0)),
        out_shape=jax.ShapeDtypeStruct((1, C), jnp.float32),
    )

    bih2 = b_ih.reshape(1, 3 * C)
    bhh2 = b_hh.reshape(1, 3 * C)
    bl2 = b_lin.reshape(1, C)
    del b_att

    h = x
    for i in range(L):
        p = sc_scatter(h, srcr, dstr)
        h = gru(p, h, weight[i], W_ih, W_hh, bih2, bhh2)
    return final(h, W_lin, bl2, W_att)


# speedup vs baseline: 1.8625x; 1.8625x over previous
import functools

import jax
import jax.numpy as jnp
from jax import lax
from jax.experimental import pallas as pl
from jax.experimental.pallas import tpu as pltpu
from jax.experimental.pallas import tpu_sc as plsc

NC = 2
NS = 16
NW = NC * NS
K = 128


def _make_sc_scatter(N, C, n_pad, nch):
    rpt = n_pad // NS
    zch = rpt // 8

    mesh = plsc.VectorSubcoreMesh(core_axis_name="c", subcore_axis_name="s")

    @functools.partial(
        pl.kernel,
        mesh=mesh,
        out_type=jax.ShapeDtypeStruct((NC, n_pad, C), jnp.float32),
        scratch_types=[
            pltpu.VMEM((nch, K), jnp.int32),
            pltpu.VMEM((nch, K), jnp.int32),
            pltpu.VMEM((K, C), jnp.float32),
            pltpu.VMEM((zch, C), jnp.float32),
            pltpu.VMEM_SHARED((n_pad, C), jnp.float32),
        ],
    )
    def sc_scatter(h_hbm, src_hbm, dst_hbm, out_hbm,
                   src_v, dst_v, buf, zbuf, m_sh):
        c = lax.axis_index("c")
        s = lax.axis_index("s")
        wid = c * NS + s

        pltpu.sync_copy(src_hbm.at[wid], src_v)
        pltpu.sync_copy(dst_hbm.at[wid], dst_v)

        @pl.loop(0, zch)
        def _(r):
            @pl.loop(0, C, step=16)
            def _(cc):
                zbuf.at[r, pl.ds(cc, 16)][...] = jnp.zeros((16,), jnp.float32)

        @pl.loop(0, 8)
        def _(k):
            pltpu.sync_copy(zbuf, m_sh.at[pl.ds(s * rpt + k * zch, zch)])

        plsc.subcore_barrier()

        @pl.loop(0, nch)
        def _(j):
            pltpu.sync_copy(h_hbm.at[src_v.at[j]], buf)
            pltpu.sync_copy(buf, m_sh.at[dst_v.at[j]], add=True)

        plsc.subcore_barrier()
        pltpu.sync_copy(m_sh.at[pl.ds(s * rpt, rpt)],
                        out_hbm.at[c, pl.ds(s * rpt, rpt)])

    return sc_scatter


def _gru_body(p_ref, h_ref, w_ref, wih_ref, whh_ref, bih_ref, bhh_ref, o_ref):
    C = h_ref.shape[1]
    sacc = p_ref[0] + p_ref[1]
    m = jnp.dot(sacc, w_ref[...], preferred_element_type=jnp.float32)
    gi = lax.dot_general(m, wih_ref[...], (((1,), (1,)), ((), ())),
                         preferred_element_type=jnp.float32) + bih_ref[...]
    gh = lax.dot_general(h_ref[...], whh_ref[...], (((1,), (1,)), ((), ())),
                         preferred_element_type=jnp.float32) + bhh_ref[...]
    i_r, i_z, i_n = gi[:, :C], gi[:, C:2 * C], gi[:, 2 * C:]
    h_r, h_z, h_n = gh[:, :C], gh[:, C:2 * C], gh[:, 2 * C:]
    r = jax.nn.sigmoid(i_r + h_r)
    z = jax.nn.sigmoid(i_z + h_z)
    n = jnp.tanh(i_n + r * h_n)
    o_ref[...] = (1.0 - z) * n + z * h_ref[...]


def _final_body(h_ref, wl_ref, bl_ref, wa_ref, o_ref):
    hr = jnp.maximum(h_ref[...], 0.0)
    h2 = lax.dot_general(hr, wl_ref[...], (((1,), (1,)), ((), ())),
                         preferred_element_type=jnp.float32) + bl_ref[...]
    logit = lax.dot_general(h2, wa_ref[...], (((1,), (1,)), ((), ())),
                            preferred_element_type=jnp.float32)
    mx = jnp.max(logit)
    e = jnp.exp(logit - mx)
    zsum = jnp.sum(e)
    o_ref[...] = jnp.sum(e * h2, axis=0, keepdims=True) / zsum


def kernel(x, edge_index, weight, W_ih, W_hh, b_ih, b_hh, W_lin, b_lin, W_att, b_att):
    N, C = x.shape
    E = edge_index.shape[1]
    L = weight.shape[0]

    ew = -(-E // (NW * K)) * K
    e_pad = ew * NW
    nch = ew // K
    n_pad = -(-(N + 1) // (NS * 8)) * (NS * 8)

    src = edge_index[0]
    dst = edge_index[1]
    if e_pad > E:
        pad = e_pad - E
        src = jnp.concatenate([src, jnp.zeros((pad,), jnp.int32)])
        dst = jnp.concatenate([dst, jnp.full((pad,), N, jnp.int32)])
    srcr = src.reshape(NW, nch, K)
    dstr = dst.reshape(NW, nch, K)

    sc_scatter = _make_sc_scatter(N, C, n_pad, nch)

    BN = 1000
    gru = pl.pallas_call(
        _gru_body,
        grid=(N // BN,),
        in_specs=[
            pl.BlockSpec((NC, BN, C), lambda i: (0, i, 0)),
            pl.BlockSpec((BN, C), lambda i: (i, 0)),
            pl.BlockSpec((C, C), lambda i: (0, 0)),
            pl.BlockSpec((3 * C, C), lambda i: (0, 0)),
            pl.BlockSpec((3 * C, C), lambda i: (0, 0)),
            pl.BlockSpec((1, 3 * C), lambda i: (0, 0)),
            pl.BlockSpec((1, 3 * C), lambda i: (0, 0)),
        ],
        out_specs=pl.BlockSpec((BN, C), lambda i: (i, 0)),
        out_shape=jax.ShapeDtypeStruct((N, C), jnp.float32),
    )

    final = pl.pallas_call(
        _final_body,
        in_specs=[
            pl.BlockSpec((N, C), lambda: (0, 0)),
            pl.BlockSpec((C, C), lambda: (0, 0)),
            pl.BlockSpec((1, C), lambda: (0, 0)),
            pl.BlockSpec((1, C), lambda: (0, 0)),
        ],
        out_specs=pl.BlockSpec((1, C), lambda: (0, 0)),
        out_shape=jax.ShapeDtypeStruct((1, C), jnp.float32),
    )

    bih2 = b_ih.reshape(1, 3 * C)
    bhh2 = b_hh.reshape(1, 3 * C)
    bl2 = b_lin.reshape(1, C)
    del b_att

    h = x
    for i in range(L):
        p = sc_scatter(h, srcr, dstr)
        h = gru(p, h, weight[i], W_ih, W_hh, bih2, bhh2)
    return final(h, W_lin, bl2, W_att)


# speedup vs baseline: 1.9985x; 1.0730x over previous
import functools

import jax
import jax.numpy as jnp
from jax import lax
from jax.experimental import pallas as pl
from jax.experimental.pallas import tpu as pltpu
from jax.experimental.pallas import tpu_sc as plsc

NC = 2
NS = 16
NW = NC * NS
K = 128


def _make_sc_scatter(N, C, n_pad, nch0, nch1):
    rpt = n_pad // NS
    zch = rpt // 8
    nchmax = max(nch0, nch1)

    mesh = plsc.VectorSubcoreMesh(core_axis_name="c", subcore_axis_name="s")

    @functools.partial(
        pl.kernel,
        mesh=mesh,
        out_type=jax.ShapeDtypeStruct((NC, n_pad, C), jnp.float32),
        scratch_types=[
            pltpu.VMEM((nchmax, K), jnp.int32),
            pltpu.VMEM((nchmax, K), jnp.int32),
            pltpu.VMEM((K, C), jnp.float32),
            pltpu.VMEM_SHARED((n_pad, C), jnp.float32),
        ],
    )
    def sc_scatter(h_hbm, src_hbm, dst_hbm, out_hbm,
                   src_v, dst_v, buf, m_sh):
        c = lax.axis_index("c")
        s = lax.axis_index("s")
        wid = c * NS + s
        nch_t = jnp.where(c == 0, nch0, nch1)

        pltpu.sync_copy(src_hbm.at[wid], src_v)
        pltpu.sync_copy(dst_hbm.at[wid], dst_v)

        @pl.loop(0, zch)
        def _(r):
            @pl.loop(0, C, step=16)
            def _(cc):
                buf.at[r, pl.ds(cc, 16)][...] = jnp.zeros((16,), jnp.float32)

        @pl.loop(0, 8)
        def _(k):
            pltpu.sync_copy(buf.at[pl.ds(0, zch)],
                            m_sh.at[pl.ds(s * rpt + k * zch, zch)])

        plsc.subcore_barrier()

        def body(j, carry):
            pltpu.sync_copy(h_hbm.at[src_v.at[j]], buf)
            pltpu.sync_copy(buf, m_sh.at[dst_v.at[j]], add=True)
            return carry

        lax.fori_loop(0, nch_t, body, 0)

        plsc.subcore_barrier()
        pltpu.sync_copy(m_sh.at[pl.ds(s * rpt, rpt)],
                        out_hbm.at[c, pl.ds(s * rpt, rpt)])

    return sc_scatter


def _gru_body(p_ref, h_ref, w_ref, wih_ref, whh_ref, bih_ref, bhh_ref, o_ref):
    C = h_ref.shape[1]
    sacc = p_ref[0] + p_ref[1]
    m = jnp.dot(sacc, w_ref[...], preferred_element_type=jnp.float32)
    gi = lax.dot_general(m, wih_ref[...], (((1,), (1,)), ((), ())),
                         preferred_element_type=jnp.float32) + bih_ref[...]
    gh = lax.dot_general(h_ref[...], whh_ref[...], (((1,), (1,)), ((), ())),
                         preferred_element_type=jnp.float32) + bhh_ref[...]
    i_r, i_z, i_n = gi[:, :C], gi[:, C:2 * C], gi[:, 2 * C:]
    h_r, h_z, h_n = gh[:, :C], gh[:, C:2 * C], gh[:, 2 * C:]
    r = jax.nn.sigmoid(i_r + h_r)
    z = jax.nn.sigmoid(i_z + h_z)
    n = jnp.tanh(i_n + r * h_n)
    o_ref[...] = (1.0 - z) * n + z * h_ref[...]


def _final_body(h_ref, wl_ref, bl_ref, wa_ref, o_ref):
    hr = jnp.maximum(h_ref[...], 0.0)
    h2 = lax.dot_general(hr, wl_ref[...], (((1,), (1,)), ((), ())),
                         preferred_element_type=jnp.float32) + bl_ref[...]
    logit = lax.dot_general(h2, wa_ref[...], (((1,), (1,)), ((), ())),
                            preferred_element_type=jnp.float32)
    mx = jnp.max(logit)
    e = jnp.exp(logit - mx)
    zsum = jnp.sum(e)
    o_ref[...] = jnp.sum(e * h2, axis=0, keepdims=True) / zsum


def kernel(x, edge_index, weight, W_ih, W_hh, b_ih, b_hh, W_lin, b_lin, W_att, b_att):
    N, C = x.shape
    E = edge_index.shape[1]
    L = weight.shape[0]

    ew = -(-E // (NW * K)) * K
    e_pad = ew * NW
    nch = ew // K
    n_pad = -(-(N + 1) // (NS * 8)) * (NS * 8)

    nch0 = (2 * nch * 65 + 50) // 100
    nch1 = 2 * nch - nch0
    nchmax = max(nch0, nch1)

    src = edge_index[0]
    dst = edge_index[1]
    if e_pad > E:
        pad = e_pad - E
        src = jnp.concatenate([src, jnp.zeros((pad,), jnp.int32)])
        dst = jnp.concatenate([dst, jnp.full((pad,), N, jnp.int32)])

    def pack(a, fill):
        e0 = NS * nch0 * K
        a0 = a[:e0].reshape(NS, nch0, K)
        a1 = a[e0:].reshape(NS, nch1, K)
        a0 = jnp.concatenate(
            [a0, jnp.full((NS, nchmax - nch0, K), fill, jnp.int32)], axis=1)
        a1 = jnp.concatenate(
            [a1, jnp.full((NS, nchmax - nch1, K), fill, jnp.int32)], axis=1)
        return jnp.concatenate([a0, a1], axis=0)

    srcr = pack(src, 0)
    dstr = pack(dst, N)

    sc_scatter = _make_sc_scatter(N, C, n_pad, nch0, nch1)

    BN = 1000
    gru = pl.pallas_call(
        _gru_body,
        grid=(N // BN,),
        in_specs=[
            pl.BlockSpec((NC, BN, C), lambda i: (0, i, 0)),
            pl.BlockSpec((BN, C), lambda i: (i, 0)),
            pl.BlockSpec((C, C), lambda i: (0, 0)),
            pl.BlockSpec((3 * C, C), lambda i: (0, 0)),
            pl.BlockSpec((3 * C, C), lambda i: (0, 0)),
            pl.BlockSpec((1, 3 * C), lambda i: (0, 0)),
            pl.BlockSpec((1, 3 * C), lambda i: (0, 0)),
        ],
        out_specs=pl.BlockSpec((BN, C), lambda i: (i, 0)),
        out_shape=jax.ShapeDtypeStruct((N, C), jnp.float32),
    )

    final = pl.pallas_call(
        _final_body,
        in_specs=[
            pl.BlockSpec((N, C), lambda: (0, 0)),
            pl.BlockSpec((C, C), lambda: (0, 0)),
            pl.BlockSpec((1, C), lambda: (0, 0)),
            pl.BlockSpec((1, C), lambda: (0, 0)),
        ],
        out_specs=pl.BlockSpec((1, C), lambda: (0, 0)),
        out_shape=jax.ShapeDtypeStruct((1, C), jnp.float32),
    )

    bih2 = b_ih.reshape(1, 3 * C)
    bhh2 = b_hh.reshape(1, 3 * C)
    bl2 = b_lin.reshape(1, C)
    del b_att

    h = x
    for i in range(L):
        p = sc_scatter(h, srcr, dstr)
        h = gru(p, h, weight[i], W_ih, W_hh, bih2, bhh2)
    return final(h, W_lin, bl2, W_att)


# speedup vs baseline: 2.1597x; 1.0807x over previous
import functools

import jax
import jax.numpy as jnp
from jax import lax
from jax.experimental import pallas as pl
from jax.experimental.pallas import tpu as pltpu
from jax.experimental.pallas import tpu_sc as plsc

NC = 2
NS = 16
NW = NC * NS
K = 128


def _make_sc_scatter(N, C, n_pad, nch0, nch1):
    rpt = n_pad // NS
    zch = rpt // 8
    nchmax = max(nch0, nch1)

    mesh = plsc.VectorSubcoreMesh(core_axis_name="c", subcore_axis_name="s")

    @functools.partial(
        pl.kernel,
        mesh=mesh,
        out_type=jax.ShapeDtypeStruct((NC, n_pad, C), jnp.float32),
        scratch_types=[
            pltpu.VMEM((nchmax, K), jnp.int32),
            pltpu.VMEM((nchmax, K), jnp.int32),
            pltpu.VMEM((K, C), jnp.float32),
            pltpu.VMEM_SHARED((n_pad, C), jnp.float32),
        ],
    )
    def sc_scatter(h_hbm, src_hbm, dst_hbm, out_hbm,
                   src_v, dst_v, buf, m_sh):
        c = lax.axis_index("c")
        s = lax.axis_index("s")
        wid = c * NS + s
        nch_t = jnp.where(c == 0, nch0, nch1)

        pltpu.sync_copy(src_hbm.at[wid], src_v)
        pltpu.sync_copy(dst_hbm.at[wid], dst_v)

        @pl.loop(0, zch)
        def _(r):
            @pl.loop(0, C, step=16)
            def _(cc):
                buf.at[r, pl.ds(cc, 16)][...] = jnp.zeros((16,), jnp.float32)

        @pl.loop(0, 8)
        def _(k):
            pltpu.sync_copy(buf.at[pl.ds(0, zch)],
                            m_sh.at[pl.ds(s * rpt + k * zch, zch)])

        plsc.subcore_barrier()

        def body(j, carry):
            pltpu.sync_copy(h_hbm.at[src_v.at[j]], buf)
            pltpu.sync_copy(buf, m_sh.at[dst_v.at[j]], add=True)
            return carry

        lax.fori_loop(0, nch_t, body, 0)

        plsc.subcore_barrier()
        pltpu.sync_copy(m_sh.at[pl.ds(s * rpt, rpt)],
                        out_hbm.at[c, pl.ds(s * rpt, rpt)])

    return sc_scatter


def _gru_body(p_ref, h_ref, w_ref, wih_ref, whh_ref, bih_ref, bhh_ref, o_ref):
    C = h_ref.shape[1]
    sacc = p_ref[0] + p_ref[1]
    m = jnp.dot(sacc, w_ref[...], preferred_element_type=jnp.float32)
    gi = lax.dot_general(m, wih_ref[...], (((1,), (1,)), ((), ())),
                         preferred_element_type=jnp.float32) + bih_ref[...]
    gh = lax.dot_general(h_ref[...], whh_ref[...], (((1,), (1,)), ((), ())),
                         preferred_element_type=jnp.float32) + bhh_ref[...]
    i_r, i_z, i_n = gi[:, :C], gi[:, C:2 * C], gi[:, 2 * C:]
    h_r, h_z, h_n = gh[:, :C], gh[:, C:2 * C], gh[:, 2 * C:]
    r = jax.nn.sigmoid(i_r + h_r)
    z = jax.nn.sigmoid(i_z + h_z)
    n = jnp.tanh(i_n + r * h_n)
    o_ref[...] = (1.0 - z) * n + z * h_ref[...]


def _final_body(h_ref, wl_ref, bl_ref, wa_ref, o_ref):
    hr = jnp.maximum(h_ref[...], 0.0)
    h2 = lax.dot_general(hr, wl_ref[...], (((1,), (1,)), ((), ())),
                         preferred_element_type=jnp.float32) + bl_ref[...]
    logit = lax.dot_general(h2, wa_ref[...], (((1,), (1,)), ((), ())),
                            preferred_element_type=jnp.float32)
    mx = jnp.max(logit)
    e = jnp.exp(logit - mx)
    zsum = jnp.sum(e)
    o_ref[...] = jnp.sum(e * h2, axis=0, keepdims=True) / zsum


def kernel(x, edge_index, weight, W_ih, W_hh, b_ih, b_hh, W_lin, b_lin, W_att, b_att):
    N, C = x.shape
    E = edge_index.shape[1]
    L = weight.shape[0]

    ew = -(-E // (NW * K)) * K
    e_pad = ew * NW
    nch = ew // K
    n_pad = -(-(N + 1) // (NS * 8)) * (NS * 8)

    nch0 = (2 * nch * 73 + 50) // 100
    nch1 = 2 * nch - nch0
    nchmax = max(nch0, nch1)

    src = edge_index[0]
    dst = edge_index[1]
    if e_pad > E:
        pad = e_pad - E
        src = jnp.concatenate([src, jnp.zeros((pad,), jnp.int32)])
        dst = jnp.concatenate([dst, jnp.full((pad,), N, jnp.int32)])

    def pack(a, fill):
        e0 = NS * nch0 * K
        a0 = a[:e0].reshape(NS, nch0, K)
        a1 = a[e0:].reshape(NS, nch1, K)
        a0 = jnp.concatenate(
            [a0, jnp.full((NS, nchmax - nch0, K), fill, jnp.int32)], axis=1)
        a1 = jnp.concatenate(
            [a1, jnp.full((NS, nchmax - nch1, K), fill, jnp.int32)], axis=1)
        return jnp.concatenate([a0, a1], axis=0)

    srcr = pack(src, 0)
    dstr = pack(dst, N)

    sc_scatter = _make_sc_scatter(N, C, n_pad, nch0, nch1)

    BN = 1000
    gru = pl.pallas_call(
        _gru_body,
        grid=(N // BN,),
        in_specs=[
            pl.BlockSpec((NC, BN, C), lambda i: (0, i, 0)),
            pl.BlockSpec((BN, C), lambda i: (i, 0)),
            pl.BlockSpec((C, C), lambda i: (0, 0)),
            pl.BlockSpec((3 * C, C), lambda i: (0, 0)),
            pl.BlockSpec((3 * C, C), lambda i: (0, 0)),
            pl.BlockSpec((1, 3 * C), lambda i: (0, 0)),
            pl.BlockSpec((1, 3 * C), lambda i: (0, 0)),
        ],
        out_specs=pl.BlockSpec((BN, C), lambda i: (i, 0)),
        out_shape=jax.ShapeDtypeStruct((N, C), jnp.float32),
    )

    final = pl.pallas_call(
        _final_body,
        in_specs=[
            pl.BlockSpec((N, C), lambda: (0, 0)),
            pl.BlockSpec((C, C), lambda: (0, 0)),
            pl.BlockSpec((1, C), lambda: (0, 0)),
            pl.BlockSpec((1, C), lambda: (0, 0)),
        ],
        out_specs=pl.BlockSpec((1, C), lambda: (0, 0)),
        out_shape=jax.ShapeDtypeStruct((1, C), jnp.float32),
    )

    bih2 = b_ih.reshape(1, 3 * C)
    bhh2 = b_hh.reshape(1, 3 * C)
    bl2 = b_lin.reshape(1, C)
    del b_att

    h = x
    for i in range(L):
        p = sc_scatter(h, srcr, dstr)
        h = gru(p, h, weight[i], W_ih, W_hh, bih2, bhh2)
    return final(h, W_lin, bl2, W_att)
